# Initial kernel scaffold; baseline (speedup 1.0000x reference)
import jax
import jax.numpy as jnp
from jax import lax
from jax.experimental import pallas as pl
from jax.experimental.pallas import tpu as pltpu
from jax.experimental.pallas import tpu_sc as plsc

L = 16
BIN_BITS = 15
N_BINS = 1 << BIN_BITS


def _auc_body(pred_hbm, lab_hbm, wt_hbm, out_hbm, p_v, l_v, w_v, htp_v, hfp_v, o_v):
    n_tasks, n = pred_hbm.shape
    cid = lax.axis_index("c")
    sid = lax.axis_index("s")
    wid = sid * 2 + cid

    @pl.when(wid < n_tasks)
    def _():
        t = wid
        pltpu.sync_copy(pred_hbm.at[t], p_v)
        pltpu.sync_copy(lab_hbm.at[t], l_v)
        pltpu.sync_copy(wt_hbm.at[t], w_v)

        zeros = jnp.zeros((L,), jnp.float32)

        def zero_body(i, carry):
            htp_v[pl.ds(i * L, L)] = zeros
            hfp_v[pl.ds(i * L, L)] = zeros
            return carry

        lax.fori_loop(0, N_BINS // L, zero_body, 0, unroll=4)

        def scat_body(i, carry):
            stp, sfp = carry
            pv = p_v[pl.ds(i * L, L)]
            bits = lax.bitcast_convert_type(pv, jnp.int32)
            flip = jnp.int32(-2147483648) | lax.shift_right_arithmetic(bits, 31)
            ukey = lax.bitwise_xor(bits, flip)
            binv = lax.shift_right_logical(ukey, 32 - BIN_BITS)
            wv = w_v[pl.ds(i * L, L)]
            lv = l_v[pl.ds(i * L, L)]
            tp = wv * lv
            fp = wv - tp
            plsc.addupdate_scatter(htp_v, [binv], tp)
            plsc.addupdate_scatter(hfp_v, [binv], fp)
            return (stp + tp, sfp + fp)

        stp, sfp = lax.fori_loop(0, n // L, scat_body, (zeros, zeros), unroll=2)
        t_tot = jnp.sum(stp)
        f_tot = jnp.sum(sfp)

        def scan_body(j, carry):
            run, acc = carry
            tpv = htp_v[pl.ds(j * L, L)]
            fpv = hfp_v[pl.ds(j * L, L)]
            cincl = plsc.cumsum(tpv) + run
            acc = acc + fpv * ((t_tot - cincl) + 0.5 * tpv)
            return (run + jnp.sum(tpv), acc)

        run, acc = lax.fori_loop(
            0, N_BINS // L, scan_body, (jnp.float32(0.0), zeros), unroll=2
        )
        trap = jnp.sum(acc)
        fac = t_tot * f_tot
        bad = fac == 0.0
        auc = jnp.where(bad, jnp.float32(0.5), trap / jnp.where(bad, jnp.float32(1.0), fac))
        o_v[...] = jnp.full((L,), auc, jnp.float32)
        pltpu.sync_copy(o_v, out_hbm.at[t])


def kernel(n_tasks, predictions, labels, weights):
    tasks, n = predictions.shape
    run = pl.kernel(
        _auc_body,
        out_type=jax.ShapeDtypeStruct((tasks, L), jnp.float32),
        mesh=plsc.VectorSubcoreMesh(core_axis_name="c", subcore_axis_name="s"),
        scratch_types=[
            pltpu.VMEM((n,), jnp.float32),
            pltpu.VMEM((n,), jnp.float32),
            pltpu.VMEM((n,), jnp.float32),
            pltpu.VMEM((N_BINS,), jnp.float32),
            pltpu.VMEM((N_BINS,), jnp.float32),
            pltpu.VMEM((L,), jnp.float32),
        ],
    )
    out = run(predictions, labels, weights)
    return out[:, 0]


# speedup vs baseline: 10.4480x; 10.4480x over previous
import jax
import jax.numpy as jnp
from jax import lax
from jax.experimental import pallas as pl
from jax.experimental.pallas import tpu as pltpu
from jax.experimental.pallas import tpu_sc as plsc

L = 16
BIN_BITS = 15
N_BINS = 1 << BIN_BITS


def _auc_body(pred_hbm, lab_hbm, wt_hbm, out_hbm, p_v, l_v, w_v, htp_v, hfp_v, o_v):
    n_tasks, n = pred_hbm.shape
    cid = lax.axis_index("c")
    sid = lax.axis_index("s")
    wid = sid * 2 + cid

    @pl.when(wid < n_tasks)
    def _():
        t = wid
        pltpu.sync_copy(pred_hbm.at[t], p_v)
        pltpu.sync_copy(lab_hbm.at[t], l_v)
        pltpu.sync_copy(wt_hbm.at[t], w_v)

        zeros = jnp.zeros((L,), jnp.float32)

        def zero_body(i, carry):
            htp_v[pl.ds(i * L, L)] = zeros
            hfp_v[pl.ds(i * L, L)] = zeros
            return carry

        lax.fori_loop(0, N_BINS // L, zero_body, 0, unroll=4)

        def scat_body(i, carry):
            stp, sfp = carry
            pv = p_v[pl.ds(i * L, L)]
            bits = lax.bitcast_convert_type(pv, jnp.int32)
            flip = jnp.int32(-2147483648) | lax.shift_right_arithmetic(bits, 31)
            ukey = lax.bitwise_xor(bits, flip)
            binv = lax.shift_right_logical(ukey, 32 - BIN_BITS)
            wv = w_v[pl.ds(i * L, L)]
            lv = l_v[pl.ds(i * L, L)]
            tp = wv * lv
            fp = wv - tp
            plsc.addupdate_scatter(htp_v, [binv], tp)
            plsc.addupdate_scatter(hfp_v, [binv], fp)
            return (stp + tp, sfp + fp)

        stp, sfp = lax.fori_loop(0, n // L, scat_body, (zeros, zeros), unroll=2)
        t_tot = jnp.sum(stp)
        f_tot = jnp.sum(sfp)

        def scan_body(j, carry):
            run, acc = carry
            tpv = htp_v[pl.ds(j * L, L)]
            fpv = hfp_v[pl.ds(j * L, L)]
            cincl = plsc.cumsum(tpv) + run
            acc = acc + fpv * ((t_tot - cincl) + 0.5 * tpv)
            return (run + jnp.sum(tpv), acc)

        run, acc = lax.fori_loop(
            0, N_BINS // L, scan_body, (jnp.float32(0.0), zeros), unroll=2
        )
        trap_v = jnp.full((L,), jnp.sum(acc), jnp.float32)
        fac_v = jnp.full((L,), t_tot * f_tot, jnp.float32)
        bad_v = fac_v == 0.0
        auc_v = jnp.where(
            bad_v,
            jnp.full((L,), 0.5, jnp.float32),
            trap_v / jnp.where(bad_v, jnp.full((L,), 1.0, jnp.float32), fac_v),
        )
        o_v[...] = auc_v
        pltpu.sync_copy(o_v, out_hbm.at[t])


def kernel(n_tasks, predictions, labels, weights):
    tasks, n = predictions.shape
    run = pl.kernel(
        _auc_body,
        out_type=jax.ShapeDtypeStruct((tasks, L), jnp.float32),
        mesh=plsc.VectorSubcoreMesh(core_axis_name="c", subcore_axis_name="s"),
        compiler_params=pltpu.CompilerParams(needs_layout_passes=False),
        scratch_types=[
            pltpu.VMEM((n,), jnp.float32),
            pltpu.VMEM((n,), jnp.float32),
            pltpu.VMEM((n,), jnp.float32),
            pltpu.VMEM((N_BINS,), jnp.float32),
            pltpu.VMEM((N_BINS,), jnp.float32),
            pltpu.VMEM((L,), jnp.float32),
        ],
    )
    out = run(predictions, labels, weights)
    return out[:, 0]
